# baseline (speedup 1.0000x reference)
import functools

import jax
import jax.numpy as jnp
from jax import lax
from jax.experimental import pallas as pl
from jax.experimental.pallas import tpu as pltpu
from jax.experimental.pallas import tpu_sc as plsc

N = 25200
E = 504000
N_PAD = 25600
E_PAD = 524288
EPT = E_PAD // 32
NCH = EPT // 128
RPT = N_PAD // 16
GRP = 16
ROW_BLK = 1008

@functools.cache
def _mesh():
    return plsc.VectorSubcoreMesh(
        core_axis_name="c", subcore_axis_name="s", num_cores=2, num_subcores=16)


def _mlp_kernel(ew_ref, w1_ref, w2_ref, out_ref):
    t = ew_ref[...].reshape(1, -1)
    t = t @ w1_ref[...].T
    t = jnp.where(t > 0, t, jnp.exp(t) - 1.0)
    t = t @ w2_ref[...].T
    t = jnp.tanh(t)
    t = jnp.maximum(t, 0.0)
    out_ref[...] = t.reshape(-1, 1)


def _edge_mlp(edge_weight, adj_w1, adj_w2):
    return pl.pallas_call(
        _mlp_kernel,
        out_shape=jax.ShapeDtypeStruct((420, 1), jnp.float32),
    )(edge_weight, adj_w1, adj_w2)


def _dis_kernel(degs_ref, out_ref):
    deg = jnp.sum(degs_ref[...], axis=0)
    out_ref[...] = jnp.where(deg > 0, lax.rsqrt(deg), 0.0)


def _dis(deg_parts):
    return pl.pallas_call(
        _dis_kernel,
        out_shape=jax.ShapeDtypeStruct((200, 128), jnp.float32),
    )(deg_parts)


def _conv1_mm_kernel(t0, t1, t2, t3, t4, w, b, out_ref):
    acc = jnp.dot(t0[...], w[0], preferred_element_type=jnp.float32)
    acc += jnp.dot(t1[...], w[1], preferred_element_type=jnp.float32)
    acc += jnp.dot(t2[...], w[2], preferred_element_type=jnp.float32)
    acc += jnp.dot(t3[...], w[3], preferred_element_type=jnp.float32)
    acc += jnp.dot(t4[...], w[4], preferred_element_type=jnp.float32)
    out_ref[...] = jnp.maximum(acc + b[...], 0.0)


def _conv1_mm(ts, w, b):
    in_spec = pl.BlockSpec((ROW_BLK, 128), lambda i: (i, 0))
    return pl.pallas_call(
        _conv1_mm_kernel,
        grid=(N // ROW_BLK,),
        in_specs=[in_spec] * 5 + [
            pl.BlockSpec((5, 128, 256), lambda i: (0, 0, 0)),
            pl.BlockSpec((1, 256), lambda i: (0, 0)),
        ],
        out_specs=pl.BlockSpec((ROW_BLK, 256), lambda i: (i, 0)),
        out_shape=jax.ShapeDtypeStruct((N, 256), jnp.float32),
    )(*ts, w, b.reshape(1, 256))


def _proj_mm_kernel(h, w, out_ref):
    out_ref[0] = jnp.dot(h[...], w[0], preferred_element_type=jnp.float32)


def _proj_mm(h, w):
    return pl.pallas_call(
        _proj_mm_kernel,
        grid=(5, N // ROW_BLK),
        in_specs=[
            pl.BlockSpec((ROW_BLK, 256), lambda k, i: (i, 0)),
            pl.BlockSpec((1, 256, 64), lambda k, i: (k, 0, 0)),
        ],
        out_specs=pl.BlockSpec((1, ROW_BLK, 64), lambda k, i: (k, i, 0)),
        out_shape=jax.ShapeDtypeStruct((5, N, 64), jnp.float32),
    )(h, w)


def _asm128_kernel(*refs, has_sub):
    if has_sub:
        pa, pb, sub, out_ref = refs
    else:
        pa, pb, out_ref = refs
    left = pa[0] + pa[1]
    right = pb[0] + pb[1]
    res = jnp.concatenate([left, right], axis=1)
    if has_sub:
        res = res - sub[...]
    out_ref[...] = res


def _asm128(pa, pb, sub=None):
    has_sub = sub is not None
    part_spec = pl.BlockSpec((2, ROW_BLK, 64), lambda i: (0, i, 0))
    in_specs = [part_spec, part_spec]
    args = [pa, pb]
    if has_sub:
        in_specs.append(pl.BlockSpec((ROW_BLK, 128), lambda i: (i, 0)))
        args.append(sub)
    return pl.pallas_call(
        functools.partial(_asm128_kernel, has_sub=has_sub),
        grid=(N // ROW_BLK,),
        in_specs=in_specs,
        out_specs=pl.BlockSpec((ROW_BLK, 128), lambda i: (i, 0)),
        out_shape=jax.ShapeDtypeStruct((N, 128), jnp.float32),
    )(*args)


def _comb64_kernel(*refs, has_sub):
    if has_sub:
        p, yy, sub, out_ref = refs
    else:
        p, yy, out_ref = refs
    res = p[0] + p[1] + yy[0]
    if has_sub:
        res = res - sub[...]
    out_ref[...] = res


def _comb64(p, yy, ycol, sub=None):
    has_sub = sub is not None
    in_specs = [
        pl.BlockSpec((2, ROW_BLK, 64), lambda i: (0, i, 0)),
        pl.BlockSpec((1, ROW_BLK, 64), lambda i, c=ycol: (c, i, 0)),
    ]
    args = [p, yy]
    if has_sub:
        in_specs.append(pl.BlockSpec((ROW_BLK, 64), lambda i: (i, 0)))
        args.append(sub)
    return pl.pallas_call(
        functools.partial(_comb64_kernel, has_sub=has_sub),
        grid=(N // ROW_BLK,),
        in_specs=in_specs,
        out_specs=pl.BlockSpec((ROW_BLK, 64), lambda i: (i, 0)),
        out_shape=jax.ShapeDtypeStruct((N, 64), jnp.float32),
    )(*args)


def _wid():
    return lax.axis_index("c") * 16 + lax.axis_index("s")


def _deg_body(rowp, coefp, out, dpriv, idxb, cb):
    w = _wid()
    def zero(i, _):
        dpriv[pl.ds(16 * i, 16)] = jnp.zeros((16,), jnp.float32)
        return 0
    lax.fori_loop(0, N_PAD // 16, zero, 0)
    base = w * EPT
    def chunk(k, _):
        e0 = base + k * 128
        pltpu.sync_copy(rowp.at[pl.ds(e0, 128)], idxb)
        pltpu.sync_copy(coefp.at[pl.ds(e0, 128)], cb)
        def grp(g, _):
            r16 = idxb[pl.ds(16 * g, 16)]
            c16 = cb[pl.ds(16 * g, 16)]
            plsc.addupdate_scatter(dpriv, [r16], c16)
            return 0
        lax.fori_loop(0, 8, grp, 0)
        return 0
    lax.fori_loop(0, NCH, chunk, 0)
    pltpu.sync_copy(dpriv, out.at[pl.ds(w * N_PAD, N_PAD)])


@functools.cache
def _deg_call():
    return pl.kernel(
        _deg_body,
        out_type=jax.ShapeDtypeStruct((32 * N_PAD,), jnp.float32),
        mesh=_mesh(),
        compiler_params=pltpu.CompilerParams(needs_layout_passes=False, use_tc_tiling_on_sc=False),
        scratch_types=[
            pltpu.VMEM((N_PAD,), jnp.float32),
            pltpu.VMEM((128,), jnp.int32),
            pltpu.VMEM((128,), jnp.float32),
        ],
    )


def _norm_body(dis, rowp, colp, coefp, na_out, nb_out, disv, idxr, idxc, cb, na, nb):
    w = _wid()
    pltpu.sync_copy(dis, disv)
    base = w * EPT
    def chunk(k, _):
        e0 = base + k * 128
        pltpu.sync_copy(rowp.at[pl.ds(e0, 128)], idxr)
        pltpu.sync_copy(colp.at[pl.ds(e0, 128)], idxc)
        pltpu.sync_copy(coefp.at[pl.ds(e0, 128)], cb)
        def grp(g, _):
            r16 = idxr[pl.ds(16 * g, 16)]
            c16 = idxc[pl.ds(16 * g, 16)]
            w16 = cb[pl.ds(16 * g, 16)]
            dr = plsc.load_gather(disv, [r16])
            dc = plsc.load_gather(disv, [c16])
            v = -(dr * w16 * dc)
            na[pl.ds(16 * g, 16)] = v
            nb[pl.ds(16 * g, 16)] = v + v
            return 0
        lax.fori_loop(0, 8, grp, 0)
        pltpu.sync_copy(na, na_out.at[pl.ds(e0, 128)])
        pltpu.sync_copy(nb, nb_out.at[pl.ds(e0, 128)])
        return 0
    lax.fori_loop(0, NCH, chunk, 0)


@functools.cache
def _norm_call():
    return pl.kernel(
        _norm_body,
        out_type=(jax.ShapeDtypeStruct((E_PAD,), jnp.float32),
                  jax.ShapeDtypeStruct((E_PAD,), jnp.float32)),
        mesh=_mesh(),
        compiler_params=pltpu.CompilerParams(needs_layout_passes=False, use_tc_tiling_on_sc=False),
        scratch_types=[
            pltpu.VMEM((N_PAD,), jnp.float32),
            pltpu.VMEM((128,), jnp.int32),
            pltpu.VMEM((128,), jnp.int32),
            pltpu.VMEM((128,), jnp.float32),
            pltpu.VMEM((128,), jnp.float32),
            pltpu.VMEM((128,), jnp.float32),
        ],
    )


def _lx_body(v2, rowp2d, colp2d, coefp, out,
             rowb, colb, cfb, gbuf, sbuf, acc, sem, *, mult, off):
    c = lax.axis_index("c")
    s = lax.axis_index("s")
    w = c * 16 + s
    def zrow(i, _):
        for j in range(4):
            gbuf[i, pl.ds(16 * j, 16)] = jnp.zeros((16,), jnp.float32)
        return 0
    lax.fori_loop(0, 128, zrow, 0)
    def zacc(i, _):
        pltpu.sync_copy(gbuf.at[pl.ds(0, 128), :],
                        acc.at[pl.ds(s * RPT + i * 128, 128), :])
        return 0
    lax.fori_loop(0, RPT // 128, zacc, 0)
    pltpu.sync_copy(gbuf.at[pl.ds(0, RPT % 128), :],
                    acc.at[pl.ds(s * RPT + RPT - RPT % 128, RPT % 128), :])
    plsc.subcore_barrier()
    def group(gi, _):
        pltpu.sync_copy(rowp2d.at[pl.ds(w * NCH + gi * GRP, GRP), :], rowb)
        pltpu.sync_copy(colp2d.at[pl.ds(w * NCH + gi * GRP, GRP), :], colb)
        pltpu.sync_copy(coefp.at[pl.ds(w * EPT + gi * (GRP * 128), GRP * 128)], cfb)
        if mult != 1 or off != 0:
            def xform(k, _):
                def xg(g, _):
                    r16 = rowb[k, pl.ds(16 * g, 16)]
                    rowb[k, pl.ds(16 * g, 16)] = r16 * mult + off
                    return 0
                lax.fori_loop(0, 8, xg, 0)
                return 0
            lax.fori_loop(0, GRP, xform, 0)
        def chunk(k, _):
            pltpu.async_copy(v2.at[rowb.at[k]], gbuf, sem).wait()
            iota = jax.lax.iota(jnp.int32, 16)
            def scale(g, _):
                c16 = cfb[pl.ds(k * 128 + 16 * g, 16)]
                rows = iota + 16 * g
                for j in range(64):
                    jj = jnp.full((16,), j, jnp.int32)
                    v = plsc.load_gather(gbuf, [rows, jj])
                    plsc.store_scatter(sbuf, [rows, jj], v * c16)
                return 0
            lax.fori_loop(0, 8, scale, 0)
            pltpu.sync_copy(sbuf, acc.at[colb.at[k]], add=True)
            return 0
        lax.fori_loop(0, GRP, chunk, 0)
        return 0
    lax.fori_loop(0, NCH // GRP, group, 0)
    plsc.subcore_barrier()
    pltpu.sync_copy(acc.at[pl.ds(s * RPT, RPT), :],
                    out.at[pl.ds(c * N_PAD + s * RPT, RPT), :])


@functools.cache
def _make_lx(mult, off):
    return pl.kernel(
        functools.partial(_lx_body, mult=mult, off=off),
        out_type=jax.ShapeDtypeStruct((2 * N_PAD, 64), jnp.float32),
        mesh=_mesh(),
        compiler_params=pltpu.CompilerParams(needs_layout_passes=False, use_tc_tiling_on_sc=False),
        scratch_types=[
            pltpu.VMEM((GRP, 128), jnp.int32),
            pltpu.VMEM((GRP, 128), jnp.int32),
            pltpu.VMEM((GRP * 128,), jnp.float32),
            pltpu.VMEM((128, 64), jnp.float32),
            pltpu.VMEM((128, 64), jnp.float32),
            pltpu.VMEM_SHARED((N_PAD, 64), jnp.float32),
            pltpu.SemaphoreType.DMA,
        ],
    )


def _lx64(v, rowp2d, colp2d, coef):
    return _make_lx(1, 0)(v, rowp2d, colp2d, coef).reshape(2, N_PAD, 64)


def _lx128(v, rowp2d, colp2d, coef):
    v2 = v.reshape(2 * N, 64)
    pa = _make_lx(2, 0)(v2, rowp2d, colp2d, coef).reshape(2, N_PAD, 64)
    pb = _make_lx(2, 1)(v2, rowp2d, colp2d, coef).reshape(2, N_PAD, 64)
    return pa, pb


def kernel(x, edge_index, edge_weight, adj_w1, adj_w2, conv1_w, conv1_b, conv2_w, conv2_b):
    ew = _edge_mlp(edge_weight, adj_w1, adj_w2)
    reps = edge_index.shape[-1] // 420
    train_ew = jnp.tile(ew, (reps, 1))

    pad = E_PAD - E
    rowp = jnp.concatenate([edge_index[0], jnp.zeros((pad,), edge_index.dtype)])
    colp = jnp.concatenate([edge_index[1], jnp.zeros((pad,), edge_index.dtype)])
    cp = jnp.concatenate([train_ew.reshape(-1), jnp.zeros((pad,), jnp.float32)])

    deg_parts = _deg_call()(rowp, cp).reshape(32, 200, 128)
    dis = _dis(deg_parts).reshape(N_PAD)
    norm, norm2 = _norm_call()(dis, rowp, colp, cp)

    row2d = rowp.reshape(E_PAD // 128, 128)
    col2d = colp.reshape(E_PAD // 128, 128)

    t0 = x
    t1 = _asm128(*_lx128(t0, row2d, col2d, norm))
    t2 = _asm128(*_lx128(t1, row2d, col2d, norm2), t0)
    t3 = _asm128(*_lx128(t2, row2d, col2d, norm2), t1)
    t4 = _asm128(*_lx128(t3, row2d, col2d, norm2), t2)
    h = _conv1_mm([t0, t1, t2, t3, t4], conv1_w, conv1_b)

    yy = _proj_mm(h, conv2_w)
    b4 = yy[4]
    b3 = _comb64(_lx64(b4, row2d, col2d, norm2), yy, 3)
    b2 = _comb64(_lx64(b3, row2d, col2d, norm2), yy, 2, b4)
    b1 = _comb64(_lx64(b2, row2d, col2d, norm2), yy, 1, b3)
    out = _comb64(_lx64(b1, row2d, col2d, norm), yy, 0, b2) + conv2_b

    return (out, ew, train_ew)


# speedup vs baseline: 2.5913x; 2.5913x over previous
import functools

import jax
import jax.numpy as jnp
from jax import lax
from jax.experimental import pallas as pl
from jax.experimental.pallas import tpu as pltpu
from jax.experimental.pallas import tpu_sc as plsc

N = 25200
E = 504000
N_PAD = 25600
E_PAD = 524288
EPT = E_PAD // 32
NCH = EPT // 128
RPT = N_PAD // 16
GRP = 8
ROW_BLK = 1008

@functools.cache
def _mesh():
    return plsc.VectorSubcoreMesh(
        core_axis_name="c", subcore_axis_name="s", num_cores=2, num_subcores=16)


def _mlp_kernel(ew_ref, w1_ref, w2_ref, out_ref):
    t = ew_ref[...].reshape(1, -1)
    t = t @ w1_ref[...].T
    t = jnp.where(t > 0, t, jnp.exp(t) - 1.0)
    t = t @ w2_ref[...].T
    t = jnp.tanh(t)
    t = jnp.maximum(t, 0.0)
    out_ref[...] = t.reshape(-1, 1)


def _edge_mlp(edge_weight, adj_w1, adj_w2):
    return pl.pallas_call(
        _mlp_kernel,
        out_shape=jax.ShapeDtypeStruct((420, 1), jnp.float32),
    )(edge_weight, adj_w1, adj_w2)


def _dis_kernel(degs_ref, out_ref):
    deg = jnp.sum(degs_ref[...], axis=0)
    out_ref[...] = jnp.where(deg > 0, lax.rsqrt(deg), 0.0)


def _dis(deg_parts):
    return pl.pallas_call(
        _dis_kernel,
        out_shape=jax.ShapeDtypeStruct((200, 128), jnp.float32),
    )(deg_parts)


def _conv1_mm_kernel(t0, t1, t2, t3, t4, w, b, out_ref):
    acc = jnp.dot(t0[...], w[0], preferred_element_type=jnp.float32)
    acc += jnp.dot(t1[...], w[1], preferred_element_type=jnp.float32)
    acc += jnp.dot(t2[...], w[2], preferred_element_type=jnp.float32)
    acc += jnp.dot(t3[...], w[3], preferred_element_type=jnp.float32)
    acc += jnp.dot(t4[...], w[4], preferred_element_type=jnp.float32)
    out_ref[...] = jnp.maximum(acc + b[...], 0.0)


def _conv1_mm(ts, w, b):
    in_spec = pl.BlockSpec((ROW_BLK, 128), lambda i: (i, 0))
    return pl.pallas_call(
        _conv1_mm_kernel,
        grid=(N // ROW_BLK,),
        in_specs=[in_spec] * 5 + [
            pl.BlockSpec((5, 128, 256), lambda i: (0, 0, 0)),
            pl.BlockSpec((1, 256), lambda i: (0, 0)),
        ],
        out_specs=pl.BlockSpec((ROW_BLK, 256), lambda i: (i, 0)),
        out_shape=jax.ShapeDtypeStruct((N, 256), jnp.float32),
    )(*ts, w, b.reshape(1, 256))


def _proj_mm_kernel(h, w, out_ref):
    out_ref[0] = jnp.dot(h[...], w[0], preferred_element_type=jnp.float32)


def _proj_mm(h, w):
    return pl.pallas_call(
        _proj_mm_kernel,
        grid=(5, N // ROW_BLK),
        in_specs=[
            pl.BlockSpec((ROW_BLK, 256), lambda k, i: (i, 0)),
            pl.BlockSpec((1, 256, 64), lambda k, i: (k, 0, 0)),
        ],
        out_specs=pl.BlockSpec((1, ROW_BLK, 64), lambda k, i: (k, i, 0)),
        out_shape=jax.ShapeDtypeStruct((5, N, 64), jnp.float32),
    )(h, w)


def _asm128_kernel(*refs, has_sub):
    if has_sub:
        pa, pb, sub, out_ref = refs
    else:
        pa, pb, out_ref = refs
    left = pa[0] + pa[1]
    right = pb[0] + pb[1]
    res = jnp.concatenate([left, right], axis=1)
    if has_sub:
        res = res - sub[...]
    out_ref[...] = res


def _asm128(pa, pb, sub=None):
    has_sub = sub is not None
    part_spec = pl.BlockSpec((2, ROW_BLK, 64), lambda i: (0, i, 0))
    in_specs = [part_spec, part_spec]
    args = [pa, pb]
    if has_sub:
        in_specs.append(pl.BlockSpec((ROW_BLK, 128), lambda i: (i, 0)))
        args.append(sub)
    return pl.pallas_call(
        functools.partial(_asm128_kernel, has_sub=has_sub),
        grid=(N // ROW_BLK,),
        in_specs=in_specs,
        out_specs=pl.BlockSpec((ROW_BLK, 128), lambda i: (i, 0)),
        out_shape=jax.ShapeDtypeStruct((N, 128), jnp.float32),
    )(*args)


def _comb64_kernel(*refs, has_sub):
    if has_sub:
        p, yy, sub, out_ref = refs
    else:
        p, yy, out_ref = refs
    res = p[0] + p[1] + yy[0]
    if has_sub:
        res = res - sub[...]
    out_ref[...] = res


def _comb64(p, yy, ycol, sub=None):
    has_sub = sub is not None
    in_specs = [
        pl.BlockSpec((2, ROW_BLK, 64), lambda i: (0, i, 0)),
        pl.BlockSpec((1, ROW_BLK, 64), lambda i, c=ycol: (c, i, 0)),
    ]
    args = [p, yy]
    if has_sub:
        in_specs.append(pl.BlockSpec((ROW_BLK, 64), lambda i: (i, 0)))
        args.append(sub)
    return pl.pallas_call(
        functools.partial(_comb64_kernel, has_sub=has_sub),
        grid=(N // ROW_BLK,),
        in_specs=in_specs,
        out_specs=pl.BlockSpec((ROW_BLK, 64), lambda i: (i, 0)),
        out_shape=jax.ShapeDtypeStruct((N, 64), jnp.float32),
    )(*args)


def _wid():
    return lax.axis_index("c") * 16 + lax.axis_index("s")


def _deg_body(rowp, coefp, out, dpriv, idxb, cb):
    w = _wid()
    def zero(i, _):
        dpriv[pl.ds(16 * i, 16)] = jnp.zeros((16,), jnp.float32)
        return 0
    lax.fori_loop(0, N_PAD // 16, zero, 0)
    base = w * EPT
    def chunk(k, _):
        e0 = base + k * 128
        pltpu.sync_copy(rowp.at[pl.ds(e0, 128)], idxb)
        pltpu.sync_copy(coefp.at[pl.ds(e0, 128)], cb)
        def grp(g, _):
            r16 = idxb[pl.ds(16 * g, 16)]
            c16 = cb[pl.ds(16 * g, 16)]
            plsc.addupdate_scatter(dpriv, [r16], c16)
            return 0
        lax.fori_loop(0, 8, grp, 0)
        return 0
    lax.fori_loop(0, NCH, chunk, 0)
    pltpu.sync_copy(dpriv, out.at[pl.ds(w * N_PAD, N_PAD)])


@functools.cache
def _deg_call():
    return pl.kernel(
        _deg_body,
        out_type=jax.ShapeDtypeStruct((32 * N_PAD,), jnp.float32),
        mesh=_mesh(),
        compiler_params=pltpu.CompilerParams(needs_layout_passes=False, use_tc_tiling_on_sc=False),
        scratch_types=[
            pltpu.VMEM((N_PAD,), jnp.float32),
            pltpu.VMEM((128,), jnp.int32),
            pltpu.VMEM((128,), jnp.float32),
        ],
    )


def _norm_body(dis, rowp, colp, coefp, na_out, nb_out, disv, idxr, idxc, cb, na, nb):
    w = _wid()
    pltpu.sync_copy(dis, disv)
    base = w * EPT
    def chunk(k, _):
        e0 = base + k * 128
        pltpu.sync_copy(rowp.at[pl.ds(e0, 128)], idxr)
        pltpu.sync_copy(colp.at[pl.ds(e0, 128)], idxc)
        pltpu.sync_copy(coefp.at[pl.ds(e0, 128)], cb)
        def grp(g, _):
            r16 = idxr[pl.ds(16 * g, 16)]
            c16 = idxc[pl.ds(16 * g, 16)]
            w16 = cb[pl.ds(16 * g, 16)]
            dr = plsc.load_gather(disv, [r16])
            dc = plsc.load_gather(disv, [c16])
            v = -(dr * w16 * dc)
            na[pl.ds(16 * g, 16)] = v
            nb[pl.ds(16 * g, 16)] = v + v
            return 0
        lax.fori_loop(0, 8, grp, 0)
        pltpu.sync_copy(na, na_out.at[pl.ds(e0, 128)])
        pltpu.sync_copy(nb, nb_out.at[pl.ds(e0, 128)])
        return 0
    lax.fori_loop(0, NCH, chunk, 0)


@functools.cache
def _norm_call():
    return pl.kernel(
        _norm_body,
        out_type=(jax.ShapeDtypeStruct((E_PAD,), jnp.float32),
                  jax.ShapeDtypeStruct((E_PAD,), jnp.float32)),
        mesh=_mesh(),
        compiler_params=pltpu.CompilerParams(needs_layout_passes=False, use_tc_tiling_on_sc=False),
        scratch_types=[
            pltpu.VMEM((N_PAD,), jnp.float32),
            pltpu.VMEM((128,), jnp.int32),
            pltpu.VMEM((128,), jnp.int32),
            pltpu.VMEM((128,), jnp.float32),
            pltpu.VMEM((128,), jnp.float32),
            pltpu.VMEM((128,), jnp.float32),
        ],
    )


def _lx_body(v2, rowp2d, colp2d, coefp, out,
             rowb, colb, cfb, g0, g1, sbuf, acc, sem0, sem1, *, mult, off):
    c = lax.axis_index("c")
    s = lax.axis_index("s")
    w = c * 16 + s
    def zrow(i, _):
        for j in range(4):
            sbuf[i, pl.ds(16 * j, 16)] = jnp.zeros((16,), jnp.float32)
        return 0
    lax.fori_loop(0, 128, zrow, 0)
    def zacc(i, _):
        pltpu.sync_copy(sbuf.at[pl.ds(0, 128), :],
                        acc.at[pl.ds(s * RPT + i * 128, 128), :])
        return 0
    lax.fori_loop(0, RPT // 128, zacc, 0)
    pltpu.sync_copy(sbuf.at[pl.ds(0, RPT % 128), :],
                    acc.at[pl.ds(s * RPT + RPT - RPT % 128, RPT % 128), :])
    plsc.subcore_barrier()

    def do_chunk(k, gbuf):
        def scale(g, _):
            for l in range(16):
                i = 16 * g + l
                cv = plsc.load_gather(cfb, [jnp.full((16,), k * 128 + i, jnp.int32)])
                for j in range(4):
                    sbuf[i, pl.ds(16 * j, 16)] = gbuf[i, pl.ds(16 * j, 16)] * cv
            return 0
        lax.fori_loop(0, 8, scale, 0)
        pltpu.sync_copy(sbuf, acc.at[colb.at[k]], add=True)

    def group(gi, _):
        pltpu.sync_copy(rowp2d.at[pl.ds(w * NCH + gi * GRP, GRP), :], rowb)
        pltpu.sync_copy(colp2d.at[pl.ds(w * NCH + gi * GRP, GRP), :], colb)
        pltpu.sync_copy(coefp.at[pl.ds(w * EPT + gi * (GRP * 128), GRP * 128)], cfb)
        if mult != 1 or off != 0:
            def xform(k, _):
                def xg(g, _):
                    r16 = rowb[k, pl.ds(16 * g, 16)]
                    rowb[k, pl.ds(16 * g, 16)] = r16 * mult + off
                    return 0
                lax.fori_loop(0, 8, xg, 0)
                return 0
            lax.fori_loop(0, GRP, xform, 0)
        pltpu.async_copy(v2.at[rowb.at[0]], g0, sem0)
        def pair(m, _):
            k0 = 2 * m
            pltpu.make_async_copy(v2.at[rowb.at[k0]], g0, sem0).wait()
            pltpu.async_copy(v2.at[rowb.at[k0 + 1]], g1, sem1)
            do_chunk(k0, g0)
            pltpu.make_async_copy(v2.at[rowb.at[k0 + 1]], g1, sem1).wait()
            @pl.when(m < GRP // 2 - 1)
            def _():
                pltpu.async_copy(v2.at[rowb.at[k0 + 2]], g0, sem0)
            do_chunk(k0 + 1, g1)
            return 0
        lax.fori_loop(0, GRP // 2, pair, 0)
        return 0
    lax.fori_loop(0, NCH // GRP, group, 0)
    plsc.subcore_barrier()
    pltpu.sync_copy(acc.at[pl.ds(s * RPT, RPT), :],
                    out.at[pl.ds(c * N_PAD + s * RPT, RPT), :])


@functools.cache
def _make_lx(mult, off):
    return pl.kernel(
        functools.partial(_lx_body, mult=mult, off=off),
        out_type=jax.ShapeDtypeStruct((2 * N_PAD, 64), jnp.float32),
        mesh=_mesh(),
        compiler_params=pltpu.CompilerParams(needs_layout_passes=False, use_tc_tiling_on_sc=False),
        scratch_types=[
            pltpu.VMEM((GRP, 128), jnp.int32),
            pltpu.VMEM((GRP, 128), jnp.int32),
            pltpu.VMEM((GRP * 128,), jnp.float32),
            pltpu.VMEM((128, 64), jnp.float32),
            pltpu.VMEM((128, 64), jnp.float32),
            pltpu.VMEM((128, 64), jnp.float32),
            pltpu.VMEM_SHARED((N_PAD, 64), jnp.float32),
            pltpu.SemaphoreType.DMA,
            pltpu.SemaphoreType.DMA,
        ],
    )


def _lx64(v, rowp2d, colp2d, coef):
    return _make_lx(1, 0)(v, rowp2d, colp2d, coef).reshape(2, N_PAD, 64)


def _lx128(v, rowp2d, colp2d, coef):
    v2 = v.reshape(2 * N, 64)
    pa = _make_lx(2, 0)(v2, rowp2d, colp2d, coef).reshape(2, N_PAD, 64)
    pb = _make_lx(2, 1)(v2, rowp2d, colp2d, coef).reshape(2, N_PAD, 64)
    return pa, pb


def kernel(x, edge_index, edge_weight, adj_w1, adj_w2, conv1_w, conv1_b, conv2_w, conv2_b):
    ew = _edge_mlp(edge_weight, adj_w1, adj_w2)
    reps = edge_index.shape[-1] // 420
    train_ew = jnp.tile(ew, (reps, 1))

    pad = E_PAD - E
    rowp = jnp.concatenate([edge_index[0], jnp.zeros((pad,), edge_index.dtype)])
    colp = jnp.concatenate([edge_index[1], jnp.zeros((pad,), edge_index.dtype)])
    cp = jnp.concatenate([train_ew.reshape(-1), jnp.zeros((pad,), jnp.float32)])

    deg_parts = _deg_call()(rowp, cp).reshape(32, 200, 128)
    dis = _dis(deg_parts).reshape(N_PAD)
    norm, norm2 = _norm_call()(dis, rowp, colp, cp)

    row2d = rowp.reshape(E_PAD // 128, 128)
    col2d = colp.reshape(E_PAD // 128, 128)

    t0 = x
    t1 = _asm128(*_lx128(t0, row2d, col2d, norm))
    t2 = _asm128(*_lx128(t1, row2d, col2d, norm2), t0)
    t3 = _asm128(*_lx128(t2, row2d, col2d, norm2), t1)
    t4 = _asm128(*_lx128(t3, row2d, col2d, norm2), t2)
    h = _conv1_mm([t0, t1, t2, t3, t4], conv1_w, conv1_b)

    yy = _proj_mm(h, conv2_w)
    b4 = yy[4]
    b3 = _comb64(_lx64(b4, row2d, col2d, norm2), yy, 3)
    b2 = _comb64(_lx64(b3, row2d, col2d, norm2), yy, 2, b4)
    b1 = _comb64(_lx64(b2, row2d, col2d, norm2), yy, 1, b3)
    out = _comb64(_lx64(b1, row2d, col2d, norm), yy, 0, b2) + conv2_b

    return (out, ew, train_ew)


# speedup vs baseline: 2.6396x; 1.0186x over previous
import functools

import jax
import jax.numpy as jnp
from jax import lax
from jax.experimental import pallas as pl
from jax.experimental.pallas import tpu as pltpu
from jax.experimental.pallas import tpu_sc as plsc

N = 25200
E = 504000
N_PAD = 25600
E_PAD = 524288
EPT = E_PAD // 32
NCH = EPT // 128
CH = 2048
NPAIR = E_PAD // CH // 2
NB = 1280
GN = N_PAD // NB

_SC_PARAMS = pltpu.CompilerParams(needs_layout_passes=False,
                                  use_tc_tiling_on_sc=False)


@functools.cache
def _mesh():
    return plsc.VectorSubcoreMesh(
        core_axis_name="c", subcore_axis_name="s", num_cores=2, num_subcores=16)


def _wid():
    return lax.axis_index("c") * 16 + lax.axis_index("s")


def _mlp_kernel(ew_ref, w1_ref, w2_ref, out_ref):
    t = ew_ref[...].reshape(1, -1)
    t = t @ w1_ref[...].T
    t = jnp.where(t > 0, t, jnp.exp(t) - 1.0)
    t = t @ w2_ref[...].T
    t = jnp.tanh(t)
    t = jnp.maximum(t, 0.0)
    out_ref[...] = t.reshape(-1, 1)


def _edge_mlp(edge_weight, adj_w1, adj_w2):
    return pl.pallas_call(
        _mlp_kernel,
        out_shape=jax.ShapeDtypeStruct((420, 1), jnp.float32),
    )(edge_weight, adj_w1, adj_w2)


def _dis_kernel(degs_ref, out_ref):
    deg = jnp.sum(degs_ref[...], axis=0)
    out_ref[...] = jnp.where(deg > 0, lax.rsqrt(deg), 0.0)


def _dis(deg_parts):
    return pl.pallas_call(
        _dis_kernel,
        out_shape=jax.ShapeDtypeStruct((200, 128), jnp.float32),
    )(deg_parts)


def _xpose_kernel(x_ref, outa_ref, outb_ref):
    xt = x_ref[...].T
    z = jnp.zeros((64, N_PAD - N), jnp.float32)
    outa_ref[...] = jnp.concatenate([xt[:64], z], axis=1)
    outb_ref[...] = jnp.concatenate([xt[64:], z], axis=1)


def _xpose(x):
    return pl.pallas_call(
        _xpose_kernel,
        out_shape=(jax.ShapeDtypeStruct((64, N_PAD), jnp.float32),
                   jax.ShapeDtypeStruct((64, N_PAD), jnp.float32)),
    )(x)


def _unpose_kernel(a_ref, b_ref, out_ref):
    out_ref[...] = a_ref[...][:, :N].T + b_ref[...].reshape(1, 64)


def _unpose(a, bias):
    return pl.pallas_call(
        _unpose_kernel,
        out_shape=jax.ShapeDtypeStruct((N, 64), jnp.float32),
    )(a, bias.reshape(64, 1))


def _comb_kernel(*refs, has_y, has_sub):
    i = 0
    a = refs[i][...]; i += 1
    if has_y:
        a = a + refs[i][0]; i += 1
    if has_sub:
        a = a - refs[i][...]; i += 1
    refs[i][...] = a


def _comb(a, y=None, ycol=None, sub=None):
    has_y, has_sub = y is not None, sub is not None
    in_specs = [pl.BlockSpec((64, NB), lambda i: (0, i))]
    args = [a]
    if has_y:
        in_specs.append(pl.BlockSpec((1, 64, NB), lambda i, c=ycol: (c, 0, i)))
        args.append(y)
    if has_sub:
        in_specs.append(pl.BlockSpec((64, NB), lambda i: (0, i)))
        args.append(sub)
    return pl.pallas_call(
        functools.partial(_comb_kernel, has_y=has_y, has_sub=has_sub),
        grid=(GN,),
        in_specs=in_specs,
        out_specs=pl.BlockSpec((64, NB), lambda i: (0, i)),
        out_shape=jax.ShapeDtypeStruct((64, N_PAD), jnp.float32),
    )(*args)


def _conv1_mm_kernel(*refs):
    ts = refs[:10]
    w, b, out_ref = refs[10], refs[11], refs[12]
    wv = w[...]
    acc = None
    for k in range(5):
        for h in range(2):
            wk = wv[k, 64 * h:64 * h + 64, :]
            d = lax.dot_general(wk, ts[2 * k + h][...],
                                (((0,), (0,)), ((), ())),
                                preferred_element_type=jnp.float32)
            acc = d if acc is None else acc + d
    out_ref[...] = jnp.maximum(acc + b[...], 0.0)


def _conv1_mm(ts, w, b):
    t_spec = pl.BlockSpec((64, NB), lambda i: (0, i))
    return pl.pallas_call(
        _conv1_mm_kernel,
        grid=(GN,),
        in_specs=[t_spec] * 10 + [
            pl.BlockSpec((5, 128, 256), lambda i: (0, 0, 0)),
            pl.BlockSpec((256, 1), lambda i: (0, 0)),
        ],
        out_specs=pl.BlockSpec((256, NB), lambda i: (0, i)),
        out_shape=jax.ShapeDtypeStruct((256, N_PAD), jnp.float32),
    )(*ts, w, b.reshape(256, 1))


def _proj_mm_kernel(h, w, out_ref):
    out_ref[0] = lax.dot_general(w[0], h[...], (((0,), (0,)), ((), ())),
                                 preferred_element_type=jnp.float32)


def _proj_mm(h, w):
    return pl.pallas_call(
        _proj_mm_kernel,
        grid=(5, GN),
        in_specs=[
            pl.BlockSpec((256, NB), lambda k, i: (0, i)),
            pl.BlockSpec((1, 256, 64), lambda k, i: (k, 0, 0)),
        ],
        out_specs=pl.BlockSpec((1, 64, NB), lambda k, i: (k, 0, i)),
        out_shape=jax.ShapeDtypeStruct((5, 64, N_PAD), jnp.float32),
    )(h, w)


def _deg_body(rowp, coefp, out, dpriv, idxb, cb):
    w = _wid()
    def zero(i, _):
        dpriv[pl.ds(16 * i, 16)] = jnp.zeros((16,), jnp.float32)
        return 0
    lax.fori_loop(0, N_PAD // 16, zero, 0)
    base = w * EPT
    def chunk(k, _):
        e0 = base + k * 128
        pltpu.sync_copy(rowp.at[pl.ds(e0, 128)], idxb)
        pltpu.sync_copy(coefp.at[pl.ds(e0, 128)], cb)
        def grp(g, _):
            r16 = idxb[pl.ds(16 * g, 16)]
            c16 = cb[pl.ds(16 * g, 16)]
            plsc.addupdate_scatter(dpriv, [r16], c16)
            return 0
        lax.fori_loop(0, 8, grp, 0)
        return 0
    lax.fori_loop(0, NCH, chunk, 0)
    pltpu.sync_copy(dpriv, out.at[pl.ds(w * N_PAD, N_PAD)])


@functools.cache
def _deg_call():
    return pl.kernel(
        _deg_body,
        out_type=jax.ShapeDtypeStruct((32 * N_PAD,), jnp.float32),
        mesh=_mesh(),
        compiler_params=_SC_PARAMS,
        scratch_types=[
            pltpu.VMEM((N_PAD,), jnp.float32),
            pltpu.VMEM((128,), jnp.int32),
            pltpu.VMEM((128,), jnp.float32),
        ],
    )


def _norm_body(dis, rowp, colp, coefp, pk_out, na_out, nb_out,
               disv, idxr, idxc, cb, pkb, na, nb):
    w = _wid()
    pltpu.sync_copy(dis, disv)
    base = w * EPT
    def chunk(k, _):
        e0 = base + k * 128
        pltpu.sync_copy(rowp.at[pl.ds(e0, 128)], idxr)
        pltpu.sync_copy(colp.at[pl.ds(e0, 128)], idxc)
        pltpu.sync_copy(coefp.at[pl.ds(e0, 128)], cb)
        def grp(g, _):
            sl = pl.ds(16 * g, 16)
            r16 = idxr[sl]
            c16 = idxc[sl]
            w16 = cb[sl]
            dr = plsc.load_gather(disv, [r16])
            dc = plsc.load_gather(disv, [c16])
            v = -(dr * w16 * dc)
            pkb[sl] = jnp.bitwise_or(r16, jnp.left_shift(c16, 16))
            na[sl] = v
            nb[sl] = v + v
            return 0
        lax.fori_loop(0, 8, grp, 0)
        pltpu.sync_copy(pkb, pk_out.at[pl.ds(e0, 128)])
        pltpu.sync_copy(na, na_out.at[pl.ds(e0, 128)])
        pltpu.sync_copy(nb, nb_out.at[pl.ds(e0, 128)])
        return 0
    lax.fori_loop(0, NCH, chunk, 0)


@functools.cache
def _norm_call():
    return pl.kernel(
        _norm_body,
        out_type=(jax.ShapeDtypeStruct((E_PAD,), jnp.int32),
                  jax.ShapeDtypeStruct((E_PAD,), jnp.float32),
                  jax.ShapeDtypeStruct((E_PAD,), jnp.float32)),
        mesh=_mesh(),
        compiler_params=_SC_PARAMS,
        scratch_types=[
            pltpu.VMEM((N_PAD,), jnp.float32),
            pltpu.VMEM((128,), jnp.int32),
            pltpu.VMEM((128,), jnp.int32),
            pltpu.VMEM((128,), jnp.float32),
            pltpu.VMEM((128,), jnp.int32),
            pltpu.VMEM((128,), jnp.float32),
            pltpu.VMEM((128,), jnp.float32),
        ],
    )


def _lxt_body(vT, pkp, coefp, out,
              vbuf, abuf, pk0, pk1, cf0, cf1, spk0, spk1, scf0, scf1):
    w = _wid()
    pltpu.sync_copy(vT.at[w], vbuf)
    def zero(i, _):
        sl = pl.ds(16 * i, 16)
        z = jnp.zeros((16,), jnp.float32)
        abuf[0, sl] = z
        abuf[1, sl] = z
        return 0
    lax.fori_loop(0, N_PAD // 16, zero, 0)

    f0 = jnp.zeros((16,), jnp.int32)
    f1 = jnp.full((16,), 1, jnp.int32)

    def issue(k, pkb, cfb, spk, scf):
        pltpu.async_copy(pkp.at[pl.ds(k * CH, CH)], pkb, spk)
        pltpu.async_copy(coefp.at[pl.ds(k * CH, CH)], cfb, scf)

    def process(k, pkb, cfb, spk, scf, more):
        pltpu.make_async_copy(pkp.at[pl.ds(k * CH, CH)], pkb, spk).wait()
        pltpu.make_async_copy(coefp.at[pl.ds(k * CH, CH)], cfb, scf).wait()
        def grp4(q, _):
            for u in range(4):
                sl = pl.ds(64 * q + 16 * u, 16)
                pk16 = pkb[sl]
                c16 = cfb[sl]
                r16 = jnp.bitwise_and(pk16, 0xFFFF)
                o16 = lax.shift_right_logical(pk16, 16)
                v0 = plsc.load_gather(vbuf, [f0, r16])
                plsc.addupdate_scatter(abuf, [f0, o16], v0 * c16)
                v1 = plsc.load_gather(vbuf, [f1, r16])
                plsc.addupdate_scatter(abuf, [f1, o16], v1 * c16)
            return 0
        lax.fori_loop(0, CH // 64, grp4, 0)
        @pl.when(more)
        def _():
            issue(k + 2, pkb, cfb, spk, scf)

    issue(0, pk0, cf0, spk0, scf0)
    issue(1, pk1, cf1, spk1, scf1)
    def pair(m, _):
        more = m < NPAIR - 1
        process(2 * m, pk0, cf0, spk0, scf0, more)
        process(2 * m + 1, pk1, cf1, spk1, scf1, more)
        return 0
    lax.fori_loop(0, NPAIR, pair, 0)
    pltpu.sync_copy(abuf, out.at[w])


@functools.cache
def _lxt_call():
    return pl.kernel(
        _lxt_body,
        out_type=jax.ShapeDtypeStruct((32, 2, N_PAD), jnp.float32),
        mesh=_mesh(),
        compiler_params=_SC_PARAMS,
        scratch_types=[
            pltpu.VMEM((2, N_PAD), jnp.float32),
            pltpu.VMEM((2, N_PAD), jnp.float32),
            pltpu.VMEM((CH,), jnp.int32),
            pltpu.VMEM((CH,), jnp.int32),
            pltpu.VMEM((CH,), jnp.float32),
            pltpu.VMEM((CH,), jnp.float32),
            pltpu.SemaphoreType.DMA,
            pltpu.SemaphoreType.DMA,
            pltpu.SemaphoreType.DMA,
            pltpu.SemaphoreType.DMA,
        ],
    )


def _lx(vT, pk, coef):
    v32 = vT.reshape(32, 2, N_PAD)
    return _lxt_call()(v32, pk, coef).reshape(64, N_PAD)


def kernel(x, edge_index, edge_weight, adj_w1, adj_w2, conv1_w, conv1_b, conv2_w, conv2_b):
    ew = _edge_mlp(edge_weight, adj_w1, adj_w2)
    reps = edge_index.shape[-1] // 420
    train_ew = jnp.tile(ew, (reps, 1))

    pad = E_PAD - E
    rowp = jnp.concatenate([edge_index[0], jnp.zeros((pad,), edge_index.dtype)])
    colp = jnp.concatenate([edge_index[1], jnp.zeros((pad,), edge_index.dtype)])
    cp = jnp.concatenate([train_ew.reshape(-1), jnp.zeros((pad,), jnp.float32)])

    deg_parts = _deg_call()(rowp, cp).reshape(32, 200, 128)
    dis = _dis(deg_parts).reshape(N_PAD)
    pk, norm, norm2 = _norm_call()(dis, rowp, colp, cp)

    t0a, t0b = _xpose(x)
    t1a = _lx(t0a, pk, norm)
    t1b = _lx(t0b, pk, norm)
    t2a = _comb(_lx(t1a, pk, norm2), sub=t0a)
    t2b = _comb(_lx(t1b, pk, norm2), sub=t0b)
    t3a = _comb(_lx(t2a, pk, norm2), sub=t1a)
    t3b = _comb(_lx(t2b, pk, norm2), sub=t1b)
    t4a = _comb(_lx(t3a, pk, norm2), sub=t2a)
    t4b = _comb(_lx(t3b, pk, norm2), sub=t2b)
    h = _conv1_mm([t0a, t0b, t1a, t1b, t2a, t2b, t3a, t3b, t4a, t4b],
                  conv1_w, conv1_b)

    yy = _proj_mm(h, conv2_w)
    b4 = yy[4]
    b3 = _comb(_lx(b4, pk, norm2), y=yy, ycol=3)
    b2 = _comb(_lx(b3, pk, norm2), y=yy, ycol=2, sub=b4)
    b1 = _comb(_lx(b2, pk, norm2), y=yy, ycol=1, sub=b3)
    outT = _comb(_lx(b1, pk, norm), y=yy, ycol=0, sub=b2)
    out = _unpose(outT, conv2_b)

    return (out, ew, train_ew)


# speedup vs baseline: 6.0607x; 2.2961x over previous
import functools

import jax
import jax.numpy as jnp
from jax import lax
from jax.experimental import pallas as pl
from jax.experimental.pallas import tpu as pltpu
from jax.experimental.pallas import tpu_sc as plsc

N = 25200
E = 504000
N_PAD = 25600
E_PAD = 524288
EPT = E_PAD // 32
NCH = EPT // 128
CH = 2048
NPAIR = E_PAD // CH // 2
NB = 1280
GN = N_PAD // NB

_SC_PARAMS = pltpu.CompilerParams(needs_layout_passes=False,
                                  use_tc_tiling_on_sc=False)


@functools.cache
def _mesh():
    return plsc.VectorSubcoreMesh(
        core_axis_name="c", subcore_axis_name="s", num_cores=2, num_subcores=16)


def _wid():
    return lax.axis_index("c") * 16 + lax.axis_index("s")


def _mlp_kernel(ew_ref, w1_ref, w2_ref, out_ref):
    t = ew_ref[...].reshape(1, -1)
    t = t @ w1_ref[...].T
    t = jnp.where(t > 0, t, jnp.exp(t) - 1.0)
    t = t @ w2_ref[...].T
    t = jnp.tanh(t)
    t = jnp.maximum(t, 0.0)
    out_ref[...] = t.reshape(-1, 1)


def _edge_mlp(edge_weight, adj_w1, adj_w2):
    return pl.pallas_call(
        _mlp_kernel,
        out_shape=jax.ShapeDtypeStruct((420, 1), jnp.float32),
    )(edge_weight, adj_w1, adj_w2)


def _dis_kernel(degs_ref, out_ref):
    deg = jnp.sum(degs_ref[...], axis=0)
    out_ref[...] = jnp.where(deg > 0, lax.rsqrt(deg), 0.0)


def _dis(deg_parts):
    return pl.pallas_call(
        _dis_kernel,
        out_shape=jax.ShapeDtypeStruct((200, 128), jnp.float32),
    )(deg_parts)


def _xpose_kernel(x_ref, outa_ref, outb_ref):
    xt = x_ref[...].T
    z = jnp.zeros((64, N_PAD - N), jnp.float32)
    outa_ref[...] = jnp.concatenate([xt[:64], z], axis=1)
    outb_ref[...] = jnp.concatenate([xt[64:], z], axis=1)


def _xpose(x):
    return pl.pallas_call(
        _xpose_kernel,
        out_shape=(jax.ShapeDtypeStruct((64, N_PAD), jnp.float32),
                   jax.ShapeDtypeStruct((64, N_PAD), jnp.float32)),
    )(x)


def _unpose_kernel(a_ref, b_ref, out_ref):
    out_ref[...] = a_ref[...][:, :N].T + b_ref[...].reshape(1, 64)


def _unpose(a, bias):
    return pl.pallas_call(
        _unpose_kernel,
        out_shape=jax.ShapeDtypeStruct((N, 64), jnp.float32),
    )(a, bias.reshape(64, 1))


def _comb_kernel(*refs, has_y, has_sub):
    i = 0
    a = refs[i][...]; i += 1
    if has_y:
        a = a + refs[i][0]; i += 1
    if has_sub:
        a = a - refs[i][...]; i += 1
    refs[i][...] = a


def _comb(a, y=None, ycol=None, sub=None):
    has_y, has_sub = y is not None, sub is not None
    in_specs = [pl.BlockSpec((64, NB), lambda i: (0, i))]
    args = [a]
    if has_y:
        in_specs.append(pl.BlockSpec((1, 64, NB), lambda i, c=ycol: (c, 0, i)))
        args.append(y)
    if has_sub:
        in_specs.append(pl.BlockSpec((64, NB), lambda i: (0, i)))
        args.append(sub)
    return pl.pallas_call(
        functools.partial(_comb_kernel, has_y=has_y, has_sub=has_sub),
        grid=(GN,),
        in_specs=in_specs,
        out_specs=pl.BlockSpec((64, NB), lambda i: (0, i)),
        out_shape=jax.ShapeDtypeStruct((64, N_PAD), jnp.float32),
    )(*args)


def _conv1_mm_kernel(*refs):
    ts = refs[:10]
    w, b, out_ref = refs[10], refs[11], refs[12]
    wv = w[...]
    acc = None
    for k in range(5):
        for h in range(2):
            wk = wv[k, 64 * h:64 * h + 64, :]
            d = lax.dot_general(wk, ts[2 * k + h][...],
                                (((0,), (0,)), ((), ())),
                                preferred_element_type=jnp.float32)
            acc = d if acc is None else acc + d
    out_ref[...] = jnp.maximum(acc + b[...], 0.0)


def _conv1_mm(ts, w, b):
    t_spec = pl.BlockSpec((64, NB), lambda i: (0, i))
    return pl.pallas_call(
        _conv1_mm_kernel,
        grid=(GN,),
        in_specs=[t_spec] * 10 + [
            pl.BlockSpec((5, 128, 256), lambda i: (0, 0, 0)),
            pl.BlockSpec((256, 1), lambda i: (0, 0)),
        ],
        out_specs=pl.BlockSpec((256, NB), lambda i: (0, i)),
        out_shape=jax.ShapeDtypeStruct((256, N_PAD), jnp.float32),
    )(*ts, w, b.reshape(256, 1))


def _proj_mm_kernel(h, w, out_ref):
    out_ref[0] = lax.dot_general(w[0], h[...], (((0,), (0,)), ((), ())),
                                 preferred_element_type=jnp.float32)


def _proj_mm(h, w):
    return pl.pallas_call(
        _proj_mm_kernel,
        grid=(5, GN),
        in_specs=[
            pl.BlockSpec((256, NB), lambda k, i: (0, i)),
            pl.BlockSpec((1, 256, 64), lambda k, i: (k, 0, 0)),
        ],
        out_specs=pl.BlockSpec((1, 64, NB), lambda k, i: (k, 0, i)),
        out_shape=jax.ShapeDtypeStruct((5, 64, N_PAD), jnp.float32),
    )(h, w)


def _deg_body(rowp, coefp, out, dpriv, idxb, cb):
    w = _wid()
    def zero(i, _):
        dpriv[pl.ds(16 * i, 16)] = jnp.zeros((16,), jnp.float32)
        return 0
    lax.fori_loop(0, N_PAD // 16, zero, 0)
    base = w * EPT
    def chunk(k, _):
        e0 = base + k * 128
        pltpu.sync_copy(rowp.at[pl.ds(e0, 128)], idxb)
        pltpu.sync_copy(coefp.at[pl.ds(e0, 128)], cb)
        def grp(g, _):
            r16 = idxb[pl.ds(16 * g, 16)]
            c16 = cb[pl.ds(16 * g, 16)]
            plsc.addupdate_scatter(dpriv, [r16], c16)
            return 0
        lax.fori_loop(0, 8, grp, 0)
        return 0
    lax.fori_loop(0, NCH, chunk, 0)
    pltpu.sync_copy(dpriv, out.at[pl.ds(w * N_PAD, N_PAD)])


@functools.cache
def _deg_call():
    return pl.kernel(
        _deg_body,
        out_type=jax.ShapeDtypeStruct((32 * N_PAD,), jnp.float32),
        mesh=_mesh(),
        compiler_params=_SC_PARAMS,
        scratch_types=[
            pltpu.VMEM((N_PAD,), jnp.float32),
            pltpu.VMEM((128,), jnp.int32),
            pltpu.VMEM((128,), jnp.float32),
        ],
    )


def _norm_body(dis, rowp, colp, coefp, pk_out, na_out, nb_out,
               disv, idxr, idxc, cb, pkb, na, nb):
    w = _wid()
    pltpu.sync_copy(dis, disv)
    base = w * EPT
    def chunk(k, _):
        e0 = base + k * 128
        pltpu.sync_copy(rowp.at[pl.ds(e0, 128)], idxr)
        pltpu.sync_copy(colp.at[pl.ds(e0, 128)], idxc)
        pltpu.sync_copy(coefp.at[pl.ds(e0, 128)], cb)
        def grp(g, _):
            sl = pl.ds(16 * g, 16)
            r16 = idxr[sl]
            c16 = idxc[sl]
            w16 = cb[sl]
            dr = plsc.load_gather(disv, [r16])
            dc = plsc.load_gather(disv, [c16])
            v = -(dr * w16 * dc)
            pkb[sl] = jnp.bitwise_or(r16, jnp.left_shift(c16, 16))
            na[sl] = v
            nb[sl] = v + v
            return 0
        lax.fori_loop(0, 8, grp, 0)
        pltpu.sync_copy(pkb, pk_out.at[pl.ds(e0, 128)])
        pltpu.sync_copy(na, na_out.at[pl.ds(e0, 128)])
        pltpu.sync_copy(nb, nb_out.at[pl.ds(e0, 128)])
        return 0
    lax.fori_loop(0, NCH, chunk, 0)


@functools.cache
def _norm_call():
    return pl.kernel(
        _norm_body,
        out_type=(jax.ShapeDtypeStruct((E_PAD,), jnp.int32),
                  jax.ShapeDtypeStruct((E_PAD,), jnp.float32),
                  jax.ShapeDtypeStruct((E_PAD,), jnp.float32)),
        mesh=_mesh(),
        compiler_params=_SC_PARAMS,
        scratch_types=[
            pltpu.VMEM((N_PAD,), jnp.float32),
            pltpu.VMEM((128,), jnp.int32),
            pltpu.VMEM((128,), jnp.int32),
            pltpu.VMEM((128,), jnp.float32),
            pltpu.VMEM((128,), jnp.int32),
            pltpu.VMEM((128,), jnp.float32),
            pltpu.VMEM((128,), jnp.float32),
        ],
    )


def _lxt_body(vT, pkp, coefp, out,
              vbuf, abuf, pk0, pk1, cf0, cf1, spk0, spk1, scf0, scf1):
    w = _wid()
    pltpu.sync_copy(vT.at[w], vbuf)
    def zero(i, _):
        sl = pl.ds(16 * i, 16)
        z = jnp.zeros((16,), jnp.float32)
        abuf[0, sl] = z
        abuf[1, sl] = z
        return 0
    lax.fori_loop(0, N_PAD // 16, zero, 0)

    f0 = jnp.zeros((16,), jnp.int32)
    f1 = jnp.full((16,), 1, jnp.int32)

    def issue(k, pkb, cfb, spk, scf):
        pltpu.async_copy(pkp.at[pl.ds(k * CH, CH)], pkb, spk)
        pltpu.async_copy(coefp.at[pl.ds(k * CH, CH)], cfb, scf)

    def process(k, pkb, cfb, spk, scf, more):
        pltpu.make_async_copy(pkp.at[pl.ds(k * CH, CH)], pkb, spk).wait()
        pltpu.make_async_copy(coefp.at[pl.ds(k * CH, CH)], cfb, scf).wait()
        @plsc.parallel_loop(0, CH, 64, unroll=2)
        def _(i):
            for u in range(4):
                sl = pl.ds(i + 16 * u, 16)
                pk16 = pkb[sl]
                c16 = cfb[sl]
                r16 = jnp.bitwise_and(pk16, 0xFFFF)
                o16 = lax.shift_right_logical(pk16, 16)
                v0 = plsc.load_gather(vbuf, [f0, r16])
                plsc.addupdate_scatter(abuf, [f0, o16], v0 * c16)
                v1 = plsc.load_gather(vbuf, [f1, r16])
                plsc.addupdate_scatter(abuf, [f1, o16], v1 * c16)
        @pl.when(more)
        def _():
            issue(k + 2, pkb, cfb, spk, scf)

    issue(0, pk0, cf0, spk0, scf0)
    issue(1, pk1, cf1, spk1, scf1)
    def pair(m, _):
        more = m < NPAIR - 1
        process(2 * m, pk0, cf0, spk0, scf0, more)
        process(2 * m + 1, pk1, cf1, spk1, scf1, more)
        return 0
    lax.fori_loop(0, NPAIR, pair, 0)
    pltpu.sync_copy(abuf, out.at[w])


@functools.cache
def _lxt_call():
    return pl.kernel(
        _lxt_body,
        out_type=jax.ShapeDtypeStruct((32, 2, N_PAD), jnp.float32),
        mesh=_mesh(),
        compiler_params=_SC_PARAMS,
        scratch_types=[
            pltpu.VMEM((2, N_PAD), jnp.float32),
            pltpu.VMEM((2, N_PAD), jnp.float32),
            pltpu.VMEM((CH,), jnp.int32),
            pltpu.VMEM((CH,), jnp.int32),
            pltpu.VMEM((CH,), jnp.float32),
            pltpu.VMEM((CH,), jnp.float32),
            pltpu.SemaphoreType.DMA,
            pltpu.SemaphoreType.DMA,
            pltpu.SemaphoreType.DMA,
            pltpu.SemaphoreType.DMA,
        ],
    )


def _lx(vT, pk, coef):
    v32 = vT.reshape(32, 2, N_PAD)
    return _lxt_call()(v32, pk, coef).reshape(64, N_PAD)


def kernel(x, edge_index, edge_weight, adj_w1, adj_w2, conv1_w, conv1_b, conv2_w, conv2_b):
    ew = _edge_mlp(edge_weight, adj_w1, adj_w2)
    reps = edge_index.shape[-1] // 420
    train_ew = jnp.tile(ew, (reps, 1))

    pad = E_PAD - E
    rowp = jnp.concatenate([edge_index[0], jnp.zeros((pad,), edge_index.dtype)])
    colp = jnp.concatenate([edge_index[1], jnp.zeros((pad,), edge_index.dtype)])
    cp = jnp.concatenate([train_ew.reshape(-1), jnp.zeros((pad,), jnp.float32)])

    deg_parts = _deg_call()(rowp, cp).reshape(32, 200, 128)
    dis = _dis(deg_parts).reshape(N_PAD)
    pk, norm, norm2 = _norm_call()(dis, rowp, colp, cp)

    t0a, t0b = _xpose(x)
    t1a = _lx(t0a, pk, norm)
    t1b = _lx(t0b, pk, norm)
    t2a = _comb(_lx(t1a, pk, norm2), sub=t0a)
    t2b = _comb(_lx(t1b, pk, norm2), sub=t0b)
    t3a = _comb(_lx(t2a, pk, norm2), sub=t1a)
    t3b = _comb(_lx(t2b, pk, norm2), sub=t1b)
    t4a = _comb(_lx(t3a, pk, norm2), sub=t2a)
    t4b = _comb(_lx(t3b, pk, norm2), sub=t2b)
    h = _conv1_mm([t0a, t0b, t1a, t1b, t2a, t2b, t3a, t3b, t4a, t4b],
                  conv1_w, conv1_b)

    yy = _proj_mm(h, conv2_w)
    b4 = yy[4]
    b3 = _comb(_lx(b4, pk, norm2), y=yy, ycol=3)
    b2 = _comb(_lx(b3, pk, norm2), y=yy, ycol=2, sub=b4)
    b1 = _comb(_lx(b2, pk, norm2), y=yy, ycol=1, sub=b3)
    outT = _comb(_lx(b1, pk, norm), y=yy, ycol=0, sub=b2)
    out = _unpose(outT, conv2_b)

    return (out, ew, train_ew)


# speedup vs baseline: 6.0877x; 1.0044x over previous
import functools

import jax
import jax.numpy as jnp
from jax import lax
from jax.experimental import pallas as pl
from jax.experimental.pallas import tpu as pltpu
from jax.experimental.pallas import tpu_sc as plsc

N = 25200
E = 504000
N_PAD = 25600
E_PAD = 524288
EPT = E_PAD // 32
NCH = EPT // 128
CH = 2048
NPAIR = E_PAD // CH // 2
NB = 1280
GN = N_PAD // NB

_SC_PARAMS = pltpu.CompilerParams(needs_layout_passes=False,
                                  use_tc_tiling_on_sc=False)


@functools.cache
def _mesh():
    return plsc.VectorSubcoreMesh(
        core_axis_name="c", subcore_axis_name="s", num_cores=2, num_subcores=16)


def _wid():
    return lax.axis_index("c") * 16 + lax.axis_index("s")


def _mlp_kernel(ew_ref, w1_ref, w2_ref, out_ref):
    t = ew_ref[...].reshape(1, -1)
    t = t @ w1_ref[...].T
    t = jnp.where(t > 0, t, jnp.exp(t) - 1.0)
    t = t @ w2_ref[...].T
    t = jnp.tanh(t)
    t = jnp.maximum(t, 0.0)
    out_ref[...] = t.reshape(-1, 1)


def _edge_mlp(edge_weight, adj_w1, adj_w2):
    return pl.pallas_call(
        _mlp_kernel,
        out_shape=jax.ShapeDtypeStruct((420, 1), jnp.float32),
    )(edge_weight, adj_w1, adj_w2)


def _dis_kernel(degs_ref, out_ref):
    deg = jnp.sum(degs_ref[...], axis=0)
    out_ref[...] = jnp.where(deg > 0, lax.rsqrt(deg), 0.0)


def _dis(deg_parts):
    return pl.pallas_call(
        _dis_kernel,
        out_shape=jax.ShapeDtypeStruct((200, 128), jnp.float32),
    )(deg_parts)


def _xpose_kernel(x_ref, outa_ref, outb_ref):
    xt = x_ref[...].T
    z = jnp.zeros((64, N_PAD - N), jnp.float32)
    outa_ref[...] = jnp.concatenate([xt[:64], z], axis=1)
    outb_ref[...] = jnp.concatenate([xt[64:], z], axis=1)


def _xpose(x):
    return pl.pallas_call(
        _xpose_kernel,
        out_shape=(jax.ShapeDtypeStruct((64, N_PAD), jnp.float32),
                   jax.ShapeDtypeStruct((64, N_PAD), jnp.float32)),
    )(x)


def _unpose_kernel(a_ref, b_ref, out_ref):
    out_ref[...] = a_ref[...][:, :N].T + b_ref[...].reshape(1, 64)


def _unpose(a, bias):
    return pl.pallas_call(
        _unpose_kernel,
        out_shape=jax.ShapeDtypeStruct((N, 64), jnp.float32),
    )(a, bias.reshape(64, 1))


def _comb_kernel(*refs, has_y, has_sub):
    i = 0
    a = refs[i][...]; i += 1
    if has_y:
        a = a + refs[i][0]; i += 1
    if has_sub:
        a = a - refs[i][...]; i += 1
    refs[i][...] = a


def _comb(a, y=None, ycol=None, sub=None):
    has_y, has_sub = y is not None, sub is not None
    in_specs = [pl.BlockSpec((64, NB), lambda i: (0, i))]
    args = [a]
    if has_y:
        in_specs.append(pl.BlockSpec((1, 64, NB), lambda i, c=ycol: (c, 0, i)))
        args.append(y)
    if has_sub:
        in_specs.append(pl.BlockSpec((64, NB), lambda i: (0, i)))
        args.append(sub)
    return pl.pallas_call(
        functools.partial(_comb_kernel, has_y=has_y, has_sub=has_sub),
        grid=(GN,),
        in_specs=in_specs,
        out_specs=pl.BlockSpec((64, NB), lambda i: (0, i)),
        out_shape=jax.ShapeDtypeStruct((64, N_PAD), jnp.float32),
    )(*args)


def _conv1_mm_kernel(*refs):
    ts = refs[:10]
    w, b, out_ref = refs[10], refs[11], refs[12]
    wv = w[...]
    acc = None
    for k in range(5):
        for h in range(2):
            wk = wv[k, 64 * h:64 * h + 64, :]
            d = lax.dot_general(wk, ts[2 * k + h][...],
                                (((0,), (0,)), ((), ())),
                                preferred_element_type=jnp.float32)
            acc = d if acc is None else acc + d
    out_ref[...] = jnp.maximum(acc + b[...], 0.0)


def _conv1_mm(ts, w, b):
    t_spec = pl.BlockSpec((64, NB), lambda i: (0, i))
    return pl.pallas_call(
        _conv1_mm_kernel,
        grid=(GN,),
        in_specs=[t_spec] * 10 + [
            pl.BlockSpec((5, 128, 256), lambda i: (0, 0, 0)),
            pl.BlockSpec((256, 1), lambda i: (0, 0)),
        ],
        out_specs=pl.BlockSpec((256, NB), lambda i: (0, i)),
        out_shape=jax.ShapeDtypeStruct((256, N_PAD), jnp.float32),
    )(*ts, w, b.reshape(256, 1))


def _proj_mm_kernel(h, w, out_ref):
    out_ref[0] = lax.dot_general(w[0], h[...], (((0,), (0,)), ((), ())),
                                 preferred_element_type=jnp.float32)


def _proj_mm(h, w):
    return pl.pallas_call(
        _proj_mm_kernel,
        grid=(5, GN),
        in_specs=[
            pl.BlockSpec((256, NB), lambda k, i: (0, i)),
            pl.BlockSpec((1, 256, 64), lambda k, i: (k, 0, 0)),
        ],
        out_specs=pl.BlockSpec((1, 64, NB), lambda k, i: (k, 0, i)),
        out_shape=jax.ShapeDtypeStruct((5, 64, N_PAD), jnp.float32),
    )(h, w)


def _deg_body(rowp, coefp, out, dpriv, idxb, cb):
    w = _wid()
    def zero(i, _):
        dpriv[pl.ds(16 * i, 16)] = jnp.zeros((16,), jnp.float32)
        return 0
    lax.fori_loop(0, N_PAD // 16, zero, 0)
    base = w * EPT
    def chunk(k, _):
        e0 = base + k * 128
        pltpu.sync_copy(rowp.at[pl.ds(e0, 128)], idxb)
        pltpu.sync_copy(coefp.at[pl.ds(e0, 128)], cb)
        @plsc.parallel_loop(0, 128, 16, unroll=4)
        def _(g):
            r16 = idxb[pl.ds(g, 16)]
            c16 = cb[pl.ds(g, 16)]
            plsc.addupdate_scatter(dpriv, [r16], c16)
        return 0
    lax.fori_loop(0, NCH, chunk, 0)
    pltpu.sync_copy(dpriv, out.at[pl.ds(w * N_PAD, N_PAD)])


@functools.cache
def _deg_call():
    return pl.kernel(
        _deg_body,
        out_type=jax.ShapeDtypeStruct((32 * N_PAD,), jnp.float32),
        mesh=_mesh(),
        compiler_params=_SC_PARAMS,
        scratch_types=[
            pltpu.VMEM((N_PAD,), jnp.float32),
            pltpu.VMEM((128,), jnp.int32),
            pltpu.VMEM((128,), jnp.float32),
        ],
    )


def _norm_body(dis, rowp, colp, coefp, pk_out, na_out, nb_out,
               disv, idxr, idxc, cb, pkb, na, nb):
    w = _wid()
    pltpu.sync_copy(dis, disv)
    base = w * EPT
    def chunk(k, _):
        e0 = base + k * 128
        pltpu.sync_copy(rowp.at[pl.ds(e0, 128)], idxr)
        pltpu.sync_copy(colp.at[pl.ds(e0, 128)], idxc)
        pltpu.sync_copy(coefp.at[pl.ds(e0, 128)], cb)
        @plsc.parallel_loop(0, 128, 16, unroll=4)
        def _(g):
            sl = pl.ds(g, 16)
            r16 = idxr[sl]
            c16 = idxc[sl]
            w16 = cb[sl]
            dr = plsc.load_gather(disv, [r16])
            dc = plsc.load_gather(disv, [c16])
            v = -(dr * w16 * dc)
            pkb[sl] = jnp.bitwise_or(r16, jnp.left_shift(c16, 16))
            na[sl] = v
            nb[sl] = v + v
        pltpu.sync_copy(pkb, pk_out.at[pl.ds(e0, 128)])
        pltpu.sync_copy(na, na_out.at[pl.ds(e0, 128)])
        pltpu.sync_copy(nb, nb_out.at[pl.ds(e0, 128)])
        return 0
    lax.fori_loop(0, NCH, chunk, 0)


@functools.cache
def _norm_call():
    return pl.kernel(
        _norm_body,
        out_type=(jax.ShapeDtypeStruct((E_PAD,), jnp.int32),
                  jax.ShapeDtypeStruct((E_PAD,), jnp.float32),
                  jax.ShapeDtypeStruct((E_PAD,), jnp.float32)),
        mesh=_mesh(),
        compiler_params=_SC_PARAMS,
        scratch_types=[
            pltpu.VMEM((N_PAD,), jnp.float32),
            pltpu.VMEM((128,), jnp.int32),
            pltpu.VMEM((128,), jnp.int32),
            pltpu.VMEM((128,), jnp.float32),
            pltpu.VMEM((128,), jnp.int32),
            pltpu.VMEM((128,), jnp.float32),
            pltpu.VMEM((128,), jnp.float32),
        ],
    )


def _lxt_body(vT, pkp, coefp, out,
              vbuf, abuf, pk0, pk1, cf0, cf1, spk0, spk1, scf0, scf1):
    w = _wid()
    pltpu.sync_copy(vT.at[w], vbuf)
    def zero(i, _):
        sl = pl.ds(16 * i, 16)
        z = jnp.zeros((16,), jnp.float32)
        abuf[0, sl] = z
        abuf[1, sl] = z
        return 0
    lax.fori_loop(0, N_PAD // 16, zero, 0)

    f0 = jnp.zeros((16,), jnp.int32)
    f1 = jnp.full((16,), 1, jnp.int32)

    def issue(k, pkb, cfb, spk, scf):
        pltpu.async_copy(pkp.at[pl.ds(k * CH, CH)], pkb, spk)
        pltpu.async_copy(coefp.at[pl.ds(k * CH, CH)], cfb, scf)

    def process(k, pkb, cfb, spk, scf, more):
        pltpu.make_async_copy(pkp.at[pl.ds(k * CH, CH)], pkb, spk).wait()
        pltpu.make_async_copy(coefp.at[pl.ds(k * CH, CH)], cfb, scf).wait()
        @plsc.parallel_loop(0, CH, 64, unroll=4)
        def _(i):
            for u in range(4):
                sl = pl.ds(i + 16 * u, 16)
                pk16 = pkb[sl]
                c16 = cfb[sl]
                r16 = jnp.bitwise_and(pk16, 0xFFFF)
                o16 = lax.shift_right_logical(pk16, 16)
                v0 = plsc.load_gather(vbuf, [f0, r16])
                plsc.addupdate_scatter(abuf, [f0, o16], v0 * c16)
                v1 = plsc.load_gather(vbuf, [f1, r16])
                plsc.addupdate_scatter(abuf, [f1, o16], v1 * c16)
        @pl.when(more)
        def _():
            issue(k + 2, pkb, cfb, spk, scf)

    issue(0, pk0, cf0, spk0, scf0)
    issue(1, pk1, cf1, spk1, scf1)
    def pair(m, _):
        more = m < NPAIR - 1
        process(2 * m, pk0, cf0, spk0, scf0, more)
        process(2 * m + 1, pk1, cf1, spk1, scf1, more)
        return 0
    lax.fori_loop(0, NPAIR, pair, 0)
    pltpu.sync_copy(abuf, out.at[w])


@functools.cache
def _lxt_call():
    return pl.kernel(
        _lxt_body,
        out_type=jax.ShapeDtypeStruct((32, 2, N_PAD), jnp.float32),
        mesh=_mesh(),
        compiler_params=_SC_PARAMS,
        scratch_types=[
            pltpu.VMEM((2, N_PAD), jnp.float32),
            pltpu.VMEM((2, N_PAD), jnp.float32),
            pltpu.VMEM((CH,), jnp.int32),
            pltpu.VMEM((CH,), jnp.int32),
            pltpu.VMEM((CH,), jnp.float32),
            pltpu.VMEM((CH,), jnp.float32),
            pltpu.SemaphoreType.DMA,
            pltpu.SemaphoreType.DMA,
            pltpu.SemaphoreType.DMA,
            pltpu.SemaphoreType.DMA,
        ],
    )


def _lx(vT, pk, coef):
    v32 = vT.reshape(32, 2, N_PAD)
    return _lxt_call()(v32, pk, coef).reshape(64, N_PAD)


def kernel(x, edge_index, edge_weight, adj_w1, adj_w2, conv1_w, conv1_b, conv2_w, conv2_b):
    ew = _edge_mlp(edge_weight, adj_w1, adj_w2)
    reps = edge_index.shape[-1] // 420
    train_ew = jnp.tile(ew, (reps, 1))

    pad = E_PAD - E
    rowp = jnp.concatenate([edge_index[0], jnp.zeros((pad,), edge_index.dtype)])
    colp = jnp.concatenate([edge_index[1], jnp.zeros((pad,), edge_index.dtype)])
    cp = jnp.concatenate([train_ew.reshape(-1), jnp.zeros((pad,), jnp.float32)])

    deg_parts = _deg_call()(rowp, cp).reshape(32, 200, 128)
    dis = _dis(deg_parts).reshape(N_PAD)
    pk, norm, norm2 = _norm_call()(dis, rowp, colp, cp)

    t0a, t0b = _xpose(x)
    t1a = _lx(t0a, pk, norm)
    t1b = _lx(t0b, pk, norm)
    t2a = _comb(_lx(t1a, pk, norm2), sub=t0a)
    t2b = _comb(_lx(t1b, pk, norm2), sub=t0b)
    t3a = _comb(_lx(t2a, pk, norm2), sub=t1a)
    t3b = _comb(_lx(t2b, pk, norm2), sub=t1b)
    t4a = _comb(_lx(t3a, pk, norm2), sub=t2a)
    t4b = _comb(_lx(t3b, pk, norm2), sub=t2b)
    h = _conv1_mm([t0a, t0b, t1a, t1b, t2a, t2b, t3a, t3b, t4a, t4b],
                  conv1_w, conv1_b)

    yy = _proj_mm(h, conv2_w)
    b4 = yy[4]
    b3 = _comb(_lx(b4, pk, norm2), y=yy, ycol=3)
    b2 = _comb(_lx(b3, pk, norm2), y=yy, ycol=2, sub=b4)
    b1 = _comb(_lx(b2, pk, norm2), y=yy, ycol=1, sub=b3)
    outT = _comb(_lx(b1, pk, norm), y=yy, ycol=0, sub=b2)
    out = _unpose(outT, conv2_b)

    return (out, ew, train_ew)
